# baseline (speedup 1.0000x reference)
import functools

import jax
import jax.numpy as jnp
from jax.experimental import pallas as pl
from jax.experimental.pallas import tpu as pltpu

DIM = 1024
E = 8
K = 2
B = 2
TS = 512


def _gelu_exact(h):
    return 0.5 * h * (1.0 + jax.lax.erf(h * 0.7071067811865476))


def _fused_kernel(x0_ref, wr_ref, br_ref, x_ref, w1_hbm, b1_hbm, w2_hbm,
                  b2_hbm, o_ref, w1s, w2s, b1s, b2s, sel_v, sel_s, sem_w,
                  sem_sel):
    b = pl.program_id(0)
    s = pl.program_id(1)

    @pl.when(jnp.logical_and(b == 0, s == 0))
    def _route_and_fetch():
        scores = jax.lax.dot_general(
            x0_ref[:, 0, :], wr_ref[...], (((1,), (0,)), ((), ())),
            preferred_element_type=jnp.float32,
            precision=jax.lax.Precision.HIGHEST,
        ) + br_ref[...]
        neg = jnp.finfo(jnp.float32).min
        i0 = jnp.argmax(scores, axis=1, keepdims=True)
        col = jax.lax.broadcasted_iota(jnp.int32, scores.shape, 1)
        masked = jnp.where(col == i0, neg, scores)
        i1 = jnp.argmax(masked, axis=1, keepdims=True)
        sel = jnp.concatenate([i0, i1], axis=1).astype(jnp.int32)
        sel_v[0:B, 0:K] = sel
        pltpu.make_async_copy(sel_v, sel_s, sem_sel).start()
        pltpu.make_async_copy(sel_v, sel_s, sem_sel).wait()
        for bb in range(B):
            for kk in range(K):
                slot = bb * K + kk
                e = sel_s[bb, kk]
                pltpu.make_async_copy(
                    w1_hbm.at[e], w1s.at[slot], sem_w.at[slot, 0]).start()
                pltpu.make_async_copy(
                    b1_hbm.at[pl.ds(e, 1)], b1s.at[slot], sem_w.at[slot, 1]
                ).start()
                pltpu.make_async_copy(
                    w2_hbm.at[e], w2s.at[slot], sem_w.at[slot, 2]).start()
                pltpu.make_async_copy(
                    b2_hbm.at[pl.ds(e, 1)], b2s.at[slot], sem_w.at[slot, 3]
                ).start()

    def _wait(slot, j, ref_hbm, ref_vmem, bias=False):
        src = ref_hbm.at[pl.ds(0, 1)] if bias else ref_hbm.at[0]
        pltpu.make_async_copy(src, ref_vmem.at[slot], sem_w.at[slot, j]).wait()

    dot = functools.partial(jnp.dot, preferred_element_type=jnp.float32,
                            precision=jax.lax.Precision.DEFAULT)
    x_blk = x_ref[0]
    s0 = b * K

    @pl.when(s == 0)
    def _w0():
        _wait(s0, 0, w1_hbm, w1s)
        _wait(s0, 1, b1_hbm, b1s, bias=True)

    h0 = _gelu_exact(dot(x_blk, w1s[s0]) + b1s[s0])

    @pl.when(s == 0)
    def _w1():
        _wait(s0, 2, w2_hbm, w2s)
        _wait(s0, 3, b2_hbm, b2s, bias=True)

    y0 = dot(h0, w2s[s0])

    @pl.when(s == 0)
    def _w2():
        _wait(s0 + 1, 0, w1_hbm, w1s)
        _wait(s0 + 1, 1, b1_hbm, b1s, bias=True)

    h1 = _gelu_exact(dot(x_blk, w1s[s0 + 1]) + b1s[s0 + 1])

    @pl.when(s == 0)
    def _w3():
        _wait(s0 + 1, 2, w2_hbm, w2s)
        _wait(s0 + 1, 3, b2_hbm, b2s, bias=True)

    y1 = dot(h1, w2s[s0 + 1])
    o_ref[0] = 0.5 * (y0 + y1 + b2s[s0] + b2s[s0 + 1])


@jax.jit
def kernel(x, Wr, br, W1, b1, W2, b2):
    S = x.shape[1]
    out = pl.pallas_call(
        _fused_kernel,
        grid=(B, S // TS),
        in_specs=[
            pl.BlockSpec((B, 8, DIM), lambda b, s: (0, 0, 0)),
            pl.BlockSpec((DIM, E), lambda b, s: (0, 0)),
            pl.BlockSpec((1, E), lambda b, s: (0, 0)),
            pl.BlockSpec((1, TS, DIM), lambda b, s: (b, s, 0)),
            pl.BlockSpec(memory_space=pltpu.HBM),
            pl.BlockSpec(memory_space=pltpu.HBM),
            pl.BlockSpec(memory_space=pltpu.HBM),
            pl.BlockSpec(memory_space=pltpu.HBM),
        ],
        out_specs=pl.BlockSpec((1, TS, DIM), lambda b, s: (b, s, 0)),
        out_shape=jax.ShapeDtypeStruct((B, S, DIM), jnp.float32),
        scratch_shapes=[
            pltpu.VMEM((B * K, DIM, DIM), jnp.float32),
            pltpu.VMEM((B * K, DIM, DIM), jnp.float32),
            pltpu.VMEM((B * K, 1, DIM), jnp.float32),
            pltpu.VMEM((B * K, 1, DIM), jnp.float32),
            pltpu.VMEM((8, 128), jnp.int32),
            pltpu.SMEM((8, 128), jnp.int32),
            pltpu.SemaphoreType.DMA((B * K, 4)),
            pltpu.SemaphoreType.DMA,
        ],
        compiler_params=pltpu.CompilerParams(
            dimension_semantics=("arbitrary", "arbitrary")),
    )(x, Wr, br.reshape(1, E), x, W1, b1, W2, b2)
    return out


# speedup vs baseline: 1.1435x; 1.1435x over previous
import functools

import jax
import jax.numpy as jnp
from jax.experimental import pallas as pl
from jax.experimental.pallas import tpu as pltpu

DIM = 1024
E = 8
K = 2
B = 2
TS = 1024


def _gelu_exact(h):
    return 0.5 * h * (1.0 + jax.lax.erf(h * 0.7071067811865476))


def _fused_kernel(x0_ref, wr_ref, br_ref, x_ref, w1_hbm, b1_hbm, w2_hbm,
                  b2_hbm, o_ref, w1s, w2s, b1s, b2s, sel_v, sel_s, sem_w,
                  sem_sel):
    b = pl.program_id(0)
    s = pl.program_id(1)

    @pl.when(jnp.logical_and(b == 0, s == 0))
    def _route_and_fetch():
        scores = jax.lax.dot_general(
            x0_ref[:, 0, :], wr_ref[...], (((1,), (0,)), ((), ())),
            preferred_element_type=jnp.float32,
            precision=jax.lax.Precision.HIGHEST,
        ) + br_ref[...]
        neg = jnp.finfo(jnp.float32).min
        i0 = jnp.argmax(scores, axis=1, keepdims=True)
        col = jax.lax.broadcasted_iota(jnp.int32, scores.shape, 1)
        masked = jnp.where(col == i0, neg, scores)
        i1 = jnp.argmax(masked, axis=1, keepdims=True)
        sel = jnp.concatenate([i0, i1], axis=1).astype(jnp.int32)
        sel_v[0:B, 0:K] = sel
        pltpu.make_async_copy(sel_v, sel_s, sem_sel).start()
        pltpu.make_async_copy(sel_v, sel_s, sem_sel).wait()
        for bb in range(B):
            for kk in range(K):
                slot = bb * K + kk
                e = sel_s[bb, kk]
                pltpu.make_async_copy(
                    w1_hbm.at[e], w1s.at[slot], sem_w.at[slot, 0]).start()
                pltpu.make_async_copy(
                    b1_hbm.at[pl.ds(e, 1)], b1s.at[slot], sem_w.at[slot, 1]
                ).start()
                pltpu.make_async_copy(
                    w2_hbm.at[e], w2s.at[slot], sem_w.at[slot, 2]).start()
                pltpu.make_async_copy(
                    b2_hbm.at[pl.ds(e, 1)], b2s.at[slot], sem_w.at[slot, 3]
                ).start()

    def _wait(slot, j, ref_hbm, ref_vmem, bias=False):
        src = ref_hbm.at[pl.ds(0, 1)] if bias else ref_hbm.at[0]
        pltpu.make_async_copy(src, ref_vmem.at[slot], sem_w.at[slot, j]).wait()

    dot = functools.partial(jnp.dot, preferred_element_type=jnp.float32,
                            precision=jax.lax.Precision.DEFAULT)
    x_blk = x_ref[0]
    s0 = b * K

    @pl.when(s == 0)
    def _w0():
        for j, (hbm, vm, bias) in enumerate([(w1_hbm, w1s, False),
                                             (b1_hbm, b1s, True),
                                             (w2_hbm, w2s, False),
                                             (b2_hbm, b2s, True)]):
            _wait(s0, j, hbm, vm, bias=bias)

    h0 = _gelu_exact(dot(x_blk, w1s[s0]) + b1s[s0])
    o_ref[0] = 0.5 * (dot(h0, w2s[s0]) + b2s[s0] + b2s[s0 + 1])

    @pl.when(s == 0)
    def _w1():
        for j, (hbm, vm, bias) in enumerate([(w1_hbm, w1s, False),
                                             (b1_hbm, b1s, True),
                                             (w2_hbm, w2s, False),
                                             (b2_hbm, b2s, True)]):
            _wait(s0 + 1, j, hbm, vm, bias=bias)

    h1 = _gelu_exact(dot(x_blk, w1s[s0 + 1]) + b1s[s0 + 1])
    o_ref[0] += 0.5 * dot(h1, w2s[s0 + 1])


@jax.jit
def kernel(x, Wr, br, W1, b1, W2, b2):
    S = x.shape[1]
    out = pl.pallas_call(
        _fused_kernel,
        grid=(B, S // TS),
        in_specs=[
            pl.BlockSpec((B, 8, DIM), lambda b, s: (0, 0, 0)),
            pl.BlockSpec((DIM, E), lambda b, s: (0, 0)),
            pl.BlockSpec((1, E), lambda b, s: (0, 0)),
            pl.BlockSpec((1, TS, DIM), lambda b, s: (b, s, 0)),
            pl.BlockSpec(memory_space=pltpu.HBM),
            pl.BlockSpec(memory_space=pltpu.HBM),
            pl.BlockSpec(memory_space=pltpu.HBM),
            pl.BlockSpec(memory_space=pltpu.HBM),
        ],
        out_specs=pl.BlockSpec((1, TS, DIM), lambda b, s: (b, s, 0)),
        out_shape=jax.ShapeDtypeStruct((B, S, DIM), jnp.float32),
        scratch_shapes=[
            pltpu.VMEM((B * K, DIM, DIM), jnp.float32),
            pltpu.VMEM((B * K, DIM, DIM), jnp.float32),
            pltpu.VMEM((B * K, 1, DIM), jnp.float32),
            pltpu.VMEM((B * K, 1, DIM), jnp.float32),
            pltpu.VMEM((8, 128), jnp.int32),
            pltpu.SMEM((8, 128), jnp.int32),
            pltpu.SemaphoreType.DMA((B * K, 4)),
            pltpu.SemaphoreType.DMA,
        ],
        compiler_params=pltpu.CompilerParams(
            dimension_semantics=("arbitrary", "arbitrary")),
    )(x, Wr, br.reshape(1, E), x, W1, b1, W2, b2)
    return out


# speedup vs baseline: 1.1699x; 1.0231x over previous
import functools

import jax
import jax.numpy as jnp
from jax.experimental import pallas as pl
from jax.experimental.pallas import tpu as pltpu

DIM = 1024
E = 8
K = 2
B = 2
TS = 1024


def _gelu_exact(h):
    return 0.5 * h * (1.0 + jax.lax.erf(h * 0.7071067811865476))


def _fused_kernel(x0_ref, wr_ref, br_ref, x_hbm, w1_hbm, b1_hbm, w2_hbm,
                  b2_hbm, o_ref, w1s, w2s, b1s, b2s, xs, sel_v, sel_s, sem_w,
                  sem_x, sem_sel):
    b = pl.program_id(0)
    s = pl.program_id(1)
    ns = pl.num_programs(1)
    t = b * ns + s

    def _xcopy(i):
        return pltpu.make_async_copy(
            x_hbm.at[i // ns, pl.ds((i % ns) * TS, TS), :],
            xs.at[i % 2], sem_x.at[i % 2])

    @pl.when(t == 0)
    def _first_x():
        _xcopy(0).start()

    @pl.when(jnp.logical_and(b == 0, s == 0))
    def _route_and_fetch():
        scores = jax.lax.dot_general(
            x0_ref[:, 0, :], wr_ref[...], (((1,), (0,)), ((), ())),
            preferred_element_type=jnp.float32,
            precision=jax.lax.Precision.HIGHEST,
        ) + br_ref[...]
        neg = jnp.finfo(jnp.float32).min
        i0 = jnp.argmax(scores, axis=1, keepdims=True)
        col = jax.lax.broadcasted_iota(jnp.int32, scores.shape, 1)
        masked = jnp.where(col == i0, neg, scores)
        i1 = jnp.argmax(masked, axis=1, keepdims=True)
        sel = jnp.concatenate([i0, i1], axis=1).astype(jnp.int32)
        sel_v[0:B, 0:K] = sel
        pltpu.make_async_copy(sel_v, sel_s, sem_sel).start()
        pltpu.make_async_copy(sel_v, sel_s, sem_sel).wait()
        for bb in range(B):
            for kk in range(K):
                slot = bb * K + kk
                e = sel_s[bb, kk]
                pltpu.make_async_copy(
                    w1_hbm.at[e], w1s.at[slot], sem_w.at[slot, 0]).start()
                pltpu.make_async_copy(
                    b1_hbm.at[pl.ds(e, 1)], b1s.at[slot], sem_w.at[slot, 1]
                ).start()
                pltpu.make_async_copy(
                    w2_hbm.at[e], w2s.at[slot], sem_w.at[slot, 2]).start()
                pltpu.make_async_copy(
                    b2_hbm.at[pl.ds(e, 1)], b2s.at[slot], sem_w.at[slot, 3]
                ).start()

    def _wait(slot, j, ref_hbm, ref_vmem, bias=False):
        src = ref_hbm.at[pl.ds(0, 1)] if bias else ref_hbm.at[0]
        pltpu.make_async_copy(src, ref_vmem.at[slot], sem_w.at[slot, j]).wait()

    @pl.when(t < B * ns - 1)
    def _next_x():
        _xcopy(t + 1).start()

    _xcopy(t).wait()

    dot = functools.partial(jnp.dot, preferred_element_type=jnp.float32,
                            precision=jax.lax.Precision.DEFAULT)
    x_blk = xs[t % 2]
    s0 = b * K

    @pl.when(s == 0)
    def _w0():
        for j, (hbm, vm, bias) in enumerate([(w1_hbm, w1s, False),
                                             (b1_hbm, b1s, True),
                                             (w2_hbm, w2s, False),
                                             (b2_hbm, b2s, True)]):
            _wait(s0, j, hbm, vm, bias=bias)

    h0 = _gelu_exact(dot(x_blk, w1s[s0]) + b1s[s0])
    o_ref[0] = 0.5 * (dot(h0, w2s[s0]) + b2s[s0] + b2s[s0 + 1])

    @pl.when(s == 0)
    def _w1():
        for j, (hbm, vm, bias) in enumerate([(w1_hbm, w1s, False),
                                             (b1_hbm, b1s, True),
                                             (w2_hbm, w2s, False),
                                             (b2_hbm, b2s, True)]):
            _wait(s0 + 1, j, hbm, vm, bias=bias)

    h1 = _gelu_exact(dot(x_blk, w1s[s0 + 1]) + b1s[s0 + 1])
    o_ref[0] += 0.5 * dot(h1, w2s[s0 + 1])


@jax.jit
def kernel(x, Wr, br, W1, b1, W2, b2):
    S = x.shape[1]
    out = pl.pallas_call(
        _fused_kernel,
        grid=(B, S // TS),
        in_specs=[
            pl.BlockSpec((B, 8, DIM), lambda b, s: (0, 0, 0)),
            pl.BlockSpec((DIM, E), lambda b, s: (0, 0)),
            pl.BlockSpec((1, E), lambda b, s: (0, 0)),
            pl.BlockSpec(memory_space=pltpu.HBM),
            pl.BlockSpec(memory_space=pltpu.HBM),
            pl.BlockSpec(memory_space=pltpu.HBM),
            pl.BlockSpec(memory_space=pltpu.HBM),
            pl.BlockSpec(memory_space=pltpu.HBM),
        ],
        out_specs=pl.BlockSpec((1, TS, DIM), lambda b, s: (b, s, 0)),
        out_shape=jax.ShapeDtypeStruct((B, S, DIM), jnp.float32),
        scratch_shapes=[
            pltpu.VMEM((B * K, DIM, DIM), jnp.float32),
            pltpu.VMEM((B * K, DIM, DIM), jnp.float32),
            pltpu.VMEM((B * K, 1, DIM), jnp.float32),
            pltpu.VMEM((B * K, 1, DIM), jnp.float32),
            pltpu.VMEM((2, TS, DIM), jnp.float32),
            pltpu.VMEM((8, 128), jnp.int32),
            pltpu.SMEM((8, 128), jnp.int32),
            pltpu.SemaphoreType.DMA((B * K, 4)),
            pltpu.SemaphoreType.DMA((2,)),
            pltpu.SemaphoreType.DMA,
        ],
        compiler_params=pltpu.CompilerParams(
            dimension_semantics=("arbitrary", "arbitrary")),
    )(x, Wr, br.reshape(1, E), x, W1, b1, W2, b2)
    return out
